# baseline (speedup 1.0000x reference)
import functools

import jax
import jax.numpy as jnp
from jax import lax
from jax.experimental import pallas as pl
from jax.experimental.pallas import tpu as pltpu
from jax.experimental.pallas import tpu_sc as plsc

N = 10000
D = 256
HALF = 128
NC = 2
NS = 16
LANES = 16

E_RAW = 160000
EPAD = 163840
EN = E_RAW + N
ENPAD = 172032

_mesh = plsc.VectorSubcoreMesh(core_axis_name="c", subcore_axis_name="s")

MAGIC = 0x5F3759DF


def _rsqrt_newton(x):
    i = lax.bitcast_convert_type(x, jnp.int32)
    i = MAGIC - lax.shift_right_arithmetic(i, 1)
    y = lax.bitcast_convert_type(i, jnp.float32)
    for _ in range(3):
        y = y * (1.5 - 0.5 * x * y * y)
    return y


@functools.partial(
    pl.kernel,
    out_type=(
        jax.ShapeDtypeStruct((EPAD,), jnp.float32),
        jax.ShapeDtypeStruct((2 * N,), jnp.float32),
    ),
    mesh=_mesh,
    scratch_types=(
        pltpu.VMEM((512,), jnp.float32),
        pltpu.VMEM((512,), jnp.float32),
        pltpu.VMEM((512,), jnp.float32),
        pltpu.VMEM((512,), jnp.float32),
        pltpu.VMEM((512,), jnp.int32),
        pltpu.VMEM((4, 128), jnp.int32),
        pltpu.VMEM((16,), jnp.float32),
        pltpu.VMEM((640,), jnp.float32),
        pltpu.VMEM_SHARED((N + 16,), jnp.float32),
    ),
)
def _sc_pre(ea_hbm, d_hbm, we_hbm, ew_hbm, deg_hbm,
            a0, a1, a2, ewv, dv1, dv2, wev, zbuf, degacc):
    cid = lax.axis_index("c")
    sid = lax.axis_index("s")

    def zstep(i, _):
        zbuf[pl.ds(i * 16, 16)] = jnp.zeros((16,), jnp.float32)
        return 0
    lax.fori_loop(0, 40, zstep, 0)

    @pl.when(sid < 15)
    def _():
        pltpu.sync_copy(zbuf, degacc.at[pl.ds(sid * 640, 640)])

    @pl.when(sid == 15)
    def _():
        pltpu.sync_copy(zbuf.at[pl.ds(0, 416)], degacc.at[pl.ds(9600, 416)])

    plsc.subcore_barrier()

    pltpu.sync_copy(we_hbm, wev)
    wvec = wev[...]
    w0 = wvec[0]
    w1 = wvec[1]
    w2 = wvec[2]
    base = (cid * NS + sid) * (EPAD // (NC * NS))

    def chunk(k, _):
        off = base + k * 512
        pltpu.sync_copy(ea_hbm.at[pl.ds(off, 512)], a0)
        pltpu.sync_copy(ea_hbm.at[pl.ds(EPAD + off, 512)], a1)
        pltpu.sync_copy(ea_hbm.at[pl.ds(2 * EPAD + off, 512)], a2)
        pltpu.sync_copy(d_hbm.at[pl.ds(off, 512)], dv1)
        for r in range(4):
            for c in range(8):
                dv2[r, pl.ds(c * 16, 16)] = dv1[pl.ds(r * 128 + c * 16, 16)]

        def cstep(j, _):
            sl = pl.ds(j * 16, 16)
            z = a0[sl] * w0 + a1[sl] * w1 + a2[sl] * w2
            ewv[sl] = 1.0 / (1.0 + jnp.exp(-z))
            return 0
        lax.fori_loop(0, 32, cstep, 0)

        for b in range(4):
            pltpu.sync_copy(ewv.at[pl.ds(b * 128, 128)],
                            degacc.at[dv2.at[b]], add=True)
        pltpu.sync_copy(ewv, ew_hbm.at[pl.ds(off, 512)])
        return 0
    lax.fori_loop(0, EPAD // (NC * NS) // 512, chunk, 0)

    plsc.subcore_barrier()

    @pl.when(sid < 15)
    def _():
        pltpu.sync_copy(degacc.at[pl.ds(sid * 640, 640)], zbuf)
        pltpu.sync_copy(zbuf, deg_hbm.at[pl.ds(cid * N + sid * 640, 640)])

    @pl.when(sid == 15)
    def _():
        pltpu.sync_copy(degacc.at[pl.ds(9600, 400)], zbuf.at[pl.ds(0, 400)])
        pltpu.sync_copy(zbuf.at[pl.ds(0, 400)],
                        deg_hbm.at[pl.ds(cid * N + 9600, 400)])


@functools.partial(
    pl.kernel,
    out_type=jax.ShapeDtypeStruct((ENPAD,), jnp.float32),
    mesh=_mesh,
    compiler_params=pltpu.CompilerParams(needs_layout_passes=False),
    scratch_types=(
        pltpu.VMEM((N,), jnp.float32),
        pltpu.VMEM((N,), jnp.float32),
        pltpu.VMEM((448,), jnp.int32),
        pltpu.VMEM((448,), jnp.int32),
        pltpu.VMEM((448,), jnp.float32),
        pltpu.VMEM((448,), jnp.float32),
    ),
)
def _sc_norm(deg_hbm, s_hbm, d_hbm, w_hbm, norm_hbm,
             degv, tmpv, sv, dv, wv, nv):
    cid = lax.axis_index("c")
    sid = lax.axis_index("s")
    pltpu.sync_copy(deg_hbm.at[pl.ds(0, N)], degv)
    pltpu.sync_copy(deg_hbm.at[pl.ds(N, N)], tmpv)

    def astep(i, _):
        sl = pl.ds(i * 16, 16)
        degv[sl] = degv[sl] + tmpv[sl] + 1.0
        return 0
    lax.fori_loop(0, N // 16, astep, 0)

    base = (cid * NS + sid) * (ENPAD // (NC * NS))

    def chunk(k, _):
        off = base + k * 448
        pltpu.sync_copy(s_hbm.at[pl.ds(off, 448)], sv)
        pltpu.sync_copy(d_hbm.at[pl.ds(off, 448)], dv)
        pltpu.sync_copy(w_hbm.at[pl.ds(off, 448)], wv)

        def cstep(j, _):
            sl = pl.ds(j * 16, 16)
            dg = plsc.load_gather(degv, [sv[sl]])
            dd = plsc.load_gather(degv, [dv[sl]])
            nv[sl] = wv[sl] * _rsqrt_newton(dg * dd)
            return 0
        lax.fori_loop(0, 28, cstep, 0)
        pltpu.sync_copy(nv, norm_hbm.at[pl.ds(off, 448)])
        return 0
    lax.fori_loop(0, ENPAD // (NC * NS) // 448, chunk, 0)


_EPT = ENPAD // NS
_SUP = _EPT // 8
_SC_ROWS = 48
_NSUB = _SUP // _SC_ROWS
_RING = 4
_NQUAD = _NSUB // _RING
_PW = HALF // 2

@functools.partial(
    pl.kernel,
    out_type=jax.ShapeDtypeStruct((2 * N, HALF), jnp.float32),
    mesh=_mesh,
    scratch_types=(
        tuple(pltpu.VMEM((_SC_ROWS, HALF), jnp.float32) for _ in range(_RING)),
        pltpu.VMEM((_SUP,), jnp.int32),
        pltpu.VMEM((_SUP,), jnp.int32),
        pltpu.VMEM((_NSUB, _SC_ROWS), jnp.int32),
        pltpu.VMEM((_SUP,), jnp.float32),
        pltpu.VMEM_SHARED((N, HALF), jnp.float32),
        tuple(pltpu.SemaphoreType.DMA for _ in range(_RING)),
        tuple(pltpu.SemaphoreType.DMA for _ in range(_RING)),
    ),
)
def _sc_layer(m_hbm, s_hbm, d_hbm, norm_hbm, agg_hbm,
              sbufs, sbig, dbig, dv2, nbig, acc, gsems, ssems):
    cid = lax.axis_index("c")
    sid = lax.axis_index("s")

    def zrow(e, _):
        for j in range(8):
            sbufs[0][e, pl.ds(j * 16, 16)] = jnp.zeros((16,), jnp.float32)
        return 0
    lax.fori_loop(0, _SC_ROWS, zrow, 0)
    roff = pl.multiple_of(sid * 624, 8)
    for h in range(13):
        pltpu.sync_copy(sbufs[0], acc.at[pl.ds(roff + h * 48, 48)])

    @pl.when(sid == 15)
    def _():
        pltpu.sync_copy(sbufs[0].at[pl.ds(0, 16)],
                        acc.at[pl.ds(roff + 624, 16)])
    plsc.subcore_barrier()

    base = sid * _EPT

    def scale(l, noff):
        sbuf = sbufs[l]

        def sgroup(g, _):
            nvec = nbig[pl.ds(noff + g * 16, 16)]
            for c in range(16):
                s = nvec[c]
                e = g * 16 + c
                for j in range(8):
                    sl = pl.ds(j * 16, 16)
                    sbuf[e, sl] = sbuf[e, sl] * s
            return 0
        lax.fori_loop(0, _SC_ROWS // 16, sgroup, 0)

    def gather(b, l):
        pltpu.async_copy(
            m_hbm.at[sbig.at[pl.ds(b * _SC_ROWS, _SC_ROWS)]], sbufs[l],
            gsems[l])

    def gwait(l):
        pltpu.make_async_copy(
            m_hbm.at[sbig.at[pl.ds(0, _SC_ROWS)]], sbufs[l], gsems[l]).wait()

    def swait(l):
        pltpu.make_async_copy(sbufs[l], acc.at[dv2.at[0]], ssems[l]).wait()

    def superchunk(s, _):
        sbase = base + s * _SUP
        pltpu.sync_copy(s_hbm.at[pl.ds(cid * ENPAD + sbase, _SUP)], sbig)
        pltpu.sync_copy(d_hbm.at[pl.ds(sbase, _SUP)], dbig)
        pltpu.sync_copy(norm_hbm.at[pl.ds(sbase, _SUP)], nbig)

        def repack(j, _):
            for c in range(_SC_ROWS // 16):
                dv2[j, pl.ds(c * 16, 16)] = dbig[
                    pl.ds(j * _SC_ROWS + c * 16, 16)]
            return 0
        lax.fori_loop(0, _NSUB, repack, 0)

        for l in range(_RING):
            gather(l, l)

        def quad(q, _):
            for l in range(_RING):
                b = _RING * q + l
                gwait(l)
                scale(l, b * _SC_ROWS)
                pltpu.async_copy(sbufs[l], acc.at[dv2.at[b]], ssems[l],
                                 add=True)

            @pl.when(q < _NQUAD - 1)
            def _():
                for l in range(_RING):
                    swait(l)
                    gather(_RING * (q + 1) + l, l)
            return 0
        lax.fori_loop(0, _NQUAD, quad, 0)

        for l in range(_RING):
            swait(l)
        return 0
    lax.fori_loop(0, _EPT // _SUP, superchunk, 0)

    plsc.subcore_barrier()
    hoff = pl.multiple_of(cid * N + sid * 624, 8)
    for h in range(13):
        b = sbufs[h % 2]
        pltpu.sync_copy(acc.at[pl.ds(roff + h * 48, 48)], b)
        pltpu.sync_copy(b, agg_hbm.at[pl.ds(hoff + h * 48, 48)])

    @pl.when(sid == 15)
    def _():
        pltpu.sync_copy(acc.at[pl.ds(roff + 624, 16)],
                        sbufs[3].at[pl.ds(0, 16)])
        pltpu.sync_copy(sbufs[3].at[pl.ds(0, 16)],
                        agg_hbm.at[pl.ds(hoff + 624, 16)])


_BN = 1000
_GRID = N // _BN


def _dot(a, b):
    return jnp.dot(a, b, precision=jax.lax.Precision.HIGHEST,
                   preferred_element_type=jnp.float32)


def _tc_input(x, win, b_in, wc0):
    def body(x_ref, w_ref, b_ref, wc_ref, o_ref):
        h = jnp.maximum(_dot(x_ref[:], w_ref[:]) + b_ref[:], 0.0)
        m = _dot(h, wc_ref[:])
        o_ref[0] = m[:, :HALF]
        o_ref[1] = m[:, HALF:]

    return pl.pallas_call(
        body,
        grid=(_GRID,),
        in_specs=[
            pl.BlockSpec((_BN, D), lambda i: (i, 0)),
            pl.BlockSpec((D, D), lambda i: (0, 0)),
            pl.BlockSpec((1, D), lambda i: (0, 0)),
            pl.BlockSpec((D, D), lambda i: (0, 0)),
        ],
        out_specs=pl.BlockSpec((2, _BN, HALF), lambda i: (0, i, 0)),
        out_shape=jax.ShapeDtypeStruct((2, N, HALF), jnp.float32),
    )(x, win, b_in, wc0)


def _tc_stats(agg):
    def body(a_ref, o_ref):
        i = pl.program_id(0)
        a = jnp.concatenate([a_ref[0], a_ref[1]], axis=1)
        s1 = jnp.sum(a, axis=0, keepdims=True)
        s2 = jnp.sum(a * a, axis=0, keepdims=True)
        blk = jnp.concatenate(
            [s1, s2, jnp.zeros((6, D), jnp.float32)], axis=0)

        @pl.when(i == 0)
        def _():
            o_ref[:] = blk

        @pl.when(i > 0)
        def _():
            o_ref[:] = o_ref[:] + blk

    return pl.pallas_call(
        body,
        grid=(_GRID,),
        in_specs=[pl.BlockSpec((2, _BN, HALF), lambda i: (0, i, 0))],
        out_specs=pl.BlockSpec((8, D), lambda i: (0, 0)),
        out_shape=jax.ShapeDtypeStruct((8, D), jnp.float32),
    )(agg)


def _bn_relu(a_ref, st_ref, g_ref, b_ref):
    a = jnp.concatenate([a_ref[0], a_ref[1]], axis=1)
    mean = st_ref[0:1, :] * (1.0 / N)
    ex2 = st_ref[1:2, :] * (1.0 / N)
    var = ex2 - mean * mean
    inv = lax.rsqrt(var + 1e-5)
    return jnp.maximum((a - mean) * inv * g_ref[:] + b_ref[:], 0.0)


def _tc_mid(agg, st, g, b, wc):
    def body(a_ref, st_ref, g_ref, b_ref, wc_ref, o_ref):
        h = _bn_relu(a_ref, st_ref, g_ref, b_ref)
        m = _dot(h, wc_ref[:])
        o_ref[0] = m[:, :HALF]
        o_ref[1] = m[:, HALF:]

    return pl.pallas_call(
        body,
        grid=(_GRID,),
        in_specs=[
            pl.BlockSpec((2, _BN, HALF), lambda i: (0, i, 0)),
            pl.BlockSpec((8, D), lambda i: (0, 0)),
            pl.BlockSpec((1, D), lambda i: (0, 0)),
            pl.BlockSpec((1, D), lambda i: (0, 0)),
            pl.BlockSpec((D, D), lambda i: (0, 0)),
        ],
        out_specs=pl.BlockSpec((2, _BN, HALF), lambda i: (0, i, 0)),
        out_shape=jax.ShapeDtypeStruct((2, N, HALF), jnp.float32),
    )(agg, st, g, b, wc)


def _tc_final(agg, st, g, b):
    def body(a_ref, st_ref, g_ref, b_ref, o_ref):
        o_ref[:] = _bn_relu(a_ref, st_ref, g_ref, b_ref)

    return pl.pallas_call(
        body,
        grid=(_GRID,),
        in_specs=[
            pl.BlockSpec((2, _BN, HALF), lambda i: (0, i, 0)),
            pl.BlockSpec((8, D), lambda i: (0, 0)),
            pl.BlockSpec((1, D), lambda i: (0, 0)),
            pl.BlockSpec((1, D), lambda i: (0, 0)),
        ],
        out_specs=pl.BlockSpec((_BN, D), lambda i: (i, 0)),
        out_shape=jax.ShapeDtypeStruct((N, D), jnp.float32),
    )(agg, st, g, b)


def kernel(x, edge_index, edge_attr, We, Win, b_in, Wc, bc, gamma, beta):
    del bc
    E = edge_index.shape[1]
    L = Wc.shape[0]
    src = edge_index[0]
    dst = edge_index[1]

    eaT = jnp.pad(edge_attr.T, ((0, 0), (0, EPAD - E))).reshape(-1)
    dpre = jnp.concatenate([dst, jnp.full((EPAD - E,), N, jnp.int32)])
    wep = jnp.pad(We[:, 0], (0, 13))
    ewp, degp = _sc_pre(eaT, dpre, wep)
    ew = ewp[:E]

    loop = jnp.arange(N, dtype=jnp.int32)
    padi = jnp.zeros((ENPAD - E - N,), jnp.int32)
    s2p = jnp.concatenate([src, loop, padi])
    d2p = jnp.concatenate([dst, loop, padi])
    w2p = jnp.concatenate(
        [ew, jnp.ones((N,), jnp.float32), jnp.zeros((ENPAD - E - N,), jnp.float32)])
    normp = _sc_norm(degp, s2p, d2p, w2p)
    hi = d2p >= N // 2
    below = jnp.cumsum(jnp.where(hi, 0, 1), dtype=jnp.int32)
    above = jnp.cumsum(jnp.where(hi, 1, 0), dtype=jnp.int32)
    pos = jnp.where(hi, below[-1] + above, below) - 1
    s2p = jnp.zeros_like(s2p).at[pos].set(s2p, unique_indices=True)
    d2p = jnp.zeros_like(d2p).at[pos].set(d2p, unique_indices=True)
    normp = jnp.zeros_like(normp).at[pos].set(normp, unique_indices=True)
    s2pp = jnp.concatenate([s2p, s2p + N])

    b2 = b_in.reshape(1, D)
    mh = _tc_input(x, Win, b2, Wc[0])
    for i in range(L):
        agg2 = _sc_layer(mh.reshape(2 * N, HALF), s2pp, d2p, normp)
        agg = agg2.reshape(2, N, HALF)
        st = _tc_stats(agg)
        g = gamma[i].reshape(1, D)
        bt = beta[i].reshape(1, D)
        if i < L - 1:
            mh = _tc_mid(agg, st, g, bt, Wc[i + 1])
        else:
            out = _tc_final(agg, st, g, bt)
    return out


# speedup vs baseline: 2.9693x; 2.9693x over previous
import functools

import jax
import jax.numpy as jnp
from jax import lax
from jax.experimental import pallas as pl
from jax.experimental.pallas import tpu as pltpu
from jax.experimental.pallas import tpu_sc as plsc

N = 10000
D = 256
HALF = 128
NC = 2
NS = 16
LANES = 16

E_RAW = 160000
EPAD = 163840
EN = E_RAW + N
ENPAD = 172032

_mesh = plsc.VectorSubcoreMesh(core_axis_name="c", subcore_axis_name="s")

MAGIC = 0x5F3759DF


def _rsqrt_newton(x):
    i = lax.bitcast_convert_type(x, jnp.int32)
    i = MAGIC - lax.shift_right_arithmetic(i, 1)
    y = lax.bitcast_convert_type(i, jnp.float32)
    for _ in range(3):
        y = y * (1.5 - 0.5 * x * y * y)
    return y


@functools.partial(
    pl.kernel,
    out_type=(
        jax.ShapeDtypeStruct((EPAD,), jnp.float32),
        jax.ShapeDtypeStruct((2 * N,), jnp.float32),
    ),
    mesh=_mesh,
    scratch_types=(
        pltpu.VMEM((512,), jnp.float32),
        pltpu.VMEM((512,), jnp.float32),
        pltpu.VMEM((512,), jnp.float32),
        pltpu.VMEM((512,), jnp.float32),
        pltpu.VMEM((512,), jnp.int32),
        pltpu.VMEM((4, 128), jnp.int32),
        pltpu.VMEM((16,), jnp.float32),
        pltpu.VMEM((640,), jnp.float32),
        pltpu.VMEM_SHARED((N + 16,), jnp.float32),
    ),
)
def _sc_pre(ea_hbm, d_hbm, we_hbm, ew_hbm, deg_hbm,
            a0, a1, a2, ewv, dv1, dv2, wev, zbuf, degacc):
    cid = lax.axis_index("c")
    sid = lax.axis_index("s")

    def zstep(i, _):
        zbuf[pl.ds(i * 16, 16)] = jnp.zeros((16,), jnp.float32)
        return 0
    lax.fori_loop(0, 40, zstep, 0)

    @pl.when(sid < 15)
    def _():
        pltpu.sync_copy(zbuf, degacc.at[pl.ds(sid * 640, 640)])

    @pl.when(sid == 15)
    def _():
        pltpu.sync_copy(zbuf.at[pl.ds(0, 416)], degacc.at[pl.ds(9600, 416)])

    plsc.subcore_barrier()

    pltpu.sync_copy(we_hbm, wev)
    wvec = wev[...]
    w0 = wvec[0]
    w1 = wvec[1]
    w2 = wvec[2]
    base = (cid * NS + sid) * (EPAD // (NC * NS))

    def chunk(k, _):
        off = base + k * 512
        pltpu.sync_copy(ea_hbm.at[pl.ds(off, 512)], a0)
        pltpu.sync_copy(ea_hbm.at[pl.ds(EPAD + off, 512)], a1)
        pltpu.sync_copy(ea_hbm.at[pl.ds(2 * EPAD + off, 512)], a2)
        pltpu.sync_copy(d_hbm.at[pl.ds(off, 512)], dv1)
        for r in range(4):
            for c in range(8):
                dv2[r, pl.ds(c * 16, 16)] = dv1[pl.ds(r * 128 + c * 16, 16)]

        def cstep(j, _):
            sl = pl.ds(j * 16, 16)
            z = a0[sl] * w0 + a1[sl] * w1 + a2[sl] * w2
            ewv[sl] = 1.0 / (1.0 + jnp.exp(-z))
            return 0
        lax.fori_loop(0, 32, cstep, 0)

        for b in range(4):
            pltpu.sync_copy(ewv.at[pl.ds(b * 128, 128)],
                            degacc.at[dv2.at[b]], add=True)
        pltpu.sync_copy(ewv, ew_hbm.at[pl.ds(off, 512)])
        return 0
    lax.fori_loop(0, EPAD // (NC * NS) // 512, chunk, 0)

    plsc.subcore_barrier()

    @pl.when(sid < 15)
    def _():
        pltpu.sync_copy(degacc.at[pl.ds(sid * 640, 640)], zbuf)
        pltpu.sync_copy(zbuf, deg_hbm.at[pl.ds(cid * N + sid * 640, 640)])

    @pl.when(sid == 15)
    def _():
        pltpu.sync_copy(degacc.at[pl.ds(9600, 400)], zbuf.at[pl.ds(0, 400)])
        pltpu.sync_copy(zbuf.at[pl.ds(0, 400)],
                        deg_hbm.at[pl.ds(cid * N + 9600, 400)])


@functools.partial(
    pl.kernel,
    out_type=jax.ShapeDtypeStruct((ENPAD,), jnp.float32),
    mesh=_mesh,
    compiler_params=pltpu.CompilerParams(needs_layout_passes=False),
    scratch_types=(
        pltpu.VMEM((N,), jnp.float32),
        pltpu.VMEM((N,), jnp.float32),
        pltpu.VMEM((448,), jnp.int32),
        pltpu.VMEM((448,), jnp.int32),
        pltpu.VMEM((448,), jnp.float32),
        pltpu.VMEM((448,), jnp.float32),
    ),
)
def _sc_norm(deg_hbm, s_hbm, d_hbm, w_hbm, norm_hbm,
             degv, tmpv, sv, dv, wv, nv):
    cid = lax.axis_index("c")
    sid = lax.axis_index("s")
    pltpu.sync_copy(deg_hbm.at[pl.ds(0, N)], degv)
    pltpu.sync_copy(deg_hbm.at[pl.ds(N, N)], tmpv)

    def astep(i, _):
        sl = pl.ds(i * 16, 16)
        degv[sl] = degv[sl] + tmpv[sl] + 1.0
        return 0
    lax.fori_loop(0, N // 16, astep, 0)

    base = (cid * NS + sid) * (ENPAD // (NC * NS))

    def chunk(k, _):
        off = base + k * 448
        pltpu.sync_copy(s_hbm.at[pl.ds(off, 448)], sv)
        pltpu.sync_copy(d_hbm.at[pl.ds(off, 448)], dv)
        pltpu.sync_copy(w_hbm.at[pl.ds(off, 448)], wv)

        def cstep(j, _):
            sl = pl.ds(j * 16, 16)
            dg = plsc.load_gather(degv, [sv[sl]])
            dd = plsc.load_gather(degv, [dv[sl]])
            nv[sl] = wv[sl] * _rsqrt_newton(dg * dd)
            return 0
        lax.fori_loop(0, 28, cstep, 0)
        pltpu.sync_copy(nv, norm_hbm.at[pl.ds(off, 448)])
        return 0
    lax.fori_loop(0, ENPAD // (NC * NS) // 448, chunk, 0)


_EPT = ENPAD // NS
_SUP = _EPT // 3
_SC_ROWS = 128
_NSUB = _SUP // _SC_ROWS
_RING = 2
_NQUAD = _NSUB // _RING

@functools.partial(
    pl.kernel,
    out_type=jax.ShapeDtypeStruct((2 * N, HALF), jnp.float32),
    mesh=_mesh,
    scratch_types=(
        tuple(pltpu.VMEM((_SC_ROWS, HALF), jnp.float32) for _ in range(_RING)),
        pltpu.VMEM((_SUP,), jnp.int32),
        pltpu.VMEM((_SUP,), jnp.int32),
        pltpu.VMEM((_NSUB, _SC_ROWS), jnp.int32),
        pltpu.VMEM((_SUP,), jnp.float32),
        pltpu.VMEM_SHARED((N, HALF), jnp.float32),
        tuple(pltpu.SemaphoreType.DMA for _ in range(_RING)),
        tuple(pltpu.SemaphoreType.DMA for _ in range(_RING)),
    ),
)
def _sc_layer(m_hbm, s_hbm, d_hbm, norm_hbm, agg_hbm,
              sbufs, sbig, dbig, dv2, nbig, acc, gsems, ssems):
    cid = lax.axis_index("c")
    sid = lax.axis_index("s")

    def zrow(e, _):
        for j in range(8):
            sbufs[0][e, pl.ds(j * 16, 16)] = jnp.zeros((16,), jnp.float32)
        return 0
    lax.fori_loop(0, _SC_ROWS, zrow, 0)
    roff = pl.multiple_of(sid * 624, 8)
    for h in range(4):
        pltpu.sync_copy(sbufs[0], acc.at[pl.ds(roff + h * 128, 128)])

    @pl.when(sid < 15)
    def _():
        pltpu.sync_copy(sbufs[0].at[pl.ds(0, 112)],
                        acc.at[pl.ds(roff + 512, 112)])

    @pl.when(sid == 15)
    def _():
        pltpu.sync_copy(sbufs[0], acc.at[pl.ds(roff + 512, 128)])
    plsc.subcore_barrier()

    base = sid * _EPT

    def scale(l, noff):
        sbuf = sbufs[l]

        def sgroup(g, _):
            nvec = nbig[pl.ds(noff + g * 16, 16)]
            for c in range(16):
                s = nvec[c]
                e = g * 16 + c
                for j in range(8):
                    sl = pl.ds(j * 16, 16)
                    sbuf[e, sl] = sbuf[e, sl] * s
            return 0
        lax.fori_loop(0, _SC_ROWS // 16, sgroup, 0)

    def gather(b, l):
        pltpu.async_copy(
            m_hbm.at[sbig.at[pl.ds(b * _SC_ROWS, _SC_ROWS)]], sbufs[l],
            gsems[l])

    def gwait(l):
        pltpu.make_async_copy(
            m_hbm.at[sbig.at[pl.ds(0, _SC_ROWS)]], sbufs[l], gsems[l]).wait()

    def swait(l):
        pltpu.make_async_copy(sbufs[l], acc.at[dv2.at[0]], ssems[l]).wait()

    def superchunk(s, _):
        sbase = base + s * _SUP
        pltpu.sync_copy(s_hbm.at[pl.ds(cid * ENPAD + sbase, _SUP)], sbig)
        pltpu.sync_copy(d_hbm.at[pl.ds(sbase, _SUP)], dbig)
        pltpu.sync_copy(norm_hbm.at[pl.ds(sbase, _SUP)], nbig)

        def repack(j, _):
            for c in range(_SC_ROWS // 16):
                dv2[j, pl.ds(c * 16, 16)] = dbig[
                    pl.ds(j * _SC_ROWS + c * 16, 16)]
            return 0
        lax.fori_loop(0, _NSUB, repack, 0)

        for l in range(_RING):
            gather(l, l)

        def quad(q, _):
            for l in range(_RING):
                b = _RING * q + l
                gwait(l)
                scale(l, b * _SC_ROWS)
                pltpu.async_copy(sbufs[l], acc.at[dv2.at[b]], ssems[l],
                                 add=True)

            @pl.when(q < _NQUAD - 1)
            def _():
                for l in range(_RING):
                    swait(l)
                    gather(_RING * (q + 1) + l, l)
            return 0
        lax.fori_loop(0, _NQUAD, quad, 0)

        for l in range(_RING):
            swait(l)
        return 0
    lax.fori_loop(0, _EPT // _SUP, superchunk, 0)

    plsc.subcore_barrier()
    hoff = pl.multiple_of(cid * N + sid * 624, 8)
    for h in range(4):
        b = sbufs[h % 2]
        pltpu.sync_copy(acc.at[pl.ds(roff + h * 128, 128)], b)
        pltpu.sync_copy(b, agg_hbm.at[pl.ds(hoff + h * 128, 128)])

    @pl.when(sid < 15)
    def _():
        pltpu.sync_copy(acc.at[pl.ds(roff + 512, 112)],
                        sbufs[1].at[pl.ds(0, 112)])
        pltpu.sync_copy(sbufs[1].at[pl.ds(0, 112)],
                        agg_hbm.at[pl.ds(hoff + 512, 112)])

    @pl.when(sid == 15)
    def _():
        pltpu.sync_copy(acc.at[pl.ds(roff + 512, 128)], sbufs[1])
        pltpu.sync_copy(sbufs[1], agg_hbm.at[pl.ds(hoff + 512, 128)])


_BN = 1000
_GRID = N // _BN


def _dot(a, b):
    return jnp.dot(a, b, precision=jax.lax.Precision.HIGHEST,
                   preferred_element_type=jnp.float32)


def _tc_input(x, win, b_in, wc0):
    def body(x_ref, w_ref, b_ref, wc_ref, o_ref):
        h = jnp.maximum(_dot(x_ref[:], w_ref[:]) + b_ref[:], 0.0)
        m = _dot(h, wc_ref[:])
        o_ref[0] = m[:, :HALF]
        o_ref[1] = m[:, HALF:]

    return pl.pallas_call(
        body,
        grid=(_GRID,),
        in_specs=[
            pl.BlockSpec((_BN, D), lambda i: (i, 0)),
            pl.BlockSpec((D, D), lambda i: (0, 0)),
            pl.BlockSpec((1, D), lambda i: (0, 0)),
            pl.BlockSpec((D, D), lambda i: (0, 0)),
        ],
        out_specs=pl.BlockSpec((2, _BN, HALF), lambda i: (0, i, 0)),
        out_shape=jax.ShapeDtypeStruct((2, N, HALF), jnp.float32),
    )(x, win, b_in, wc0)


def _tc_stats(agg):
    def body(a_ref, o_ref):
        i = pl.program_id(0)
        a = jnp.concatenate([a_ref[0], a_ref[1]], axis=1)
        s1 = jnp.sum(a, axis=0, keepdims=True)
        s2 = jnp.sum(a * a, axis=0, keepdims=True)
        blk = jnp.concatenate(
            [s1, s2, jnp.zeros((6, D), jnp.float32)], axis=0)

        @pl.when(i == 0)
        def _():
            o_ref[:] = blk

        @pl.when(i > 0)
        def _():
            o_ref[:] = o_ref[:] + blk

    return pl.pallas_call(
        body,
        grid=(_GRID,),
        in_specs=[pl.BlockSpec((2, _BN, HALF), lambda i: (0, i, 0))],
        out_specs=pl.BlockSpec((8, D), lambda i: (0, 0)),
        out_shape=jax.ShapeDtypeStruct((8, D), jnp.float32),
    )(agg)


def _bn_relu(a_ref, st_ref, g_ref, b_ref):
    a = jnp.concatenate([a_ref[0], a_ref[1]], axis=1)
    mean = st_ref[0:1, :] * (1.0 / N)
    ex2 = st_ref[1:2, :] * (1.0 / N)
    var = ex2 - mean * mean
    inv = lax.rsqrt(var + 1e-5)
    return jnp.maximum((a - mean) * inv * g_ref[:] + b_ref[:], 0.0)


def _tc_mid(agg, st, g, b, wc):
    def body(a_ref, st_ref, g_ref, b_ref, wc_ref, o_ref):
        h = _bn_relu(a_ref, st_ref, g_ref, b_ref)
        m = _dot(h, wc_ref[:])
        o_ref[0] = m[:, :HALF]
        o_ref[1] = m[:, HALF:]

    return pl.pallas_call(
        body,
        grid=(_GRID,),
        in_specs=[
            pl.BlockSpec((2, _BN, HALF), lambda i: (0, i, 0)),
            pl.BlockSpec((8, D), lambda i: (0, 0)),
            pl.BlockSpec((1, D), lambda i: (0, 0)),
            pl.BlockSpec((1, D), lambda i: (0, 0)),
            pl.BlockSpec((D, D), lambda i: (0, 0)),
        ],
        out_specs=pl.BlockSpec((2, _BN, HALF), lambda i: (0, i, 0)),
        out_shape=jax.ShapeDtypeStruct((2, N, HALF), jnp.float32),
    )(agg, st, g, b, wc)


def _tc_final(agg, st, g, b):
    def body(a_ref, st_ref, g_ref, b_ref, o_ref):
        o_ref[:] = _bn_relu(a_ref, st_ref, g_ref, b_ref)

    return pl.pallas_call(
        body,
        grid=(_GRID,),
        in_specs=[
            pl.BlockSpec((2, _BN, HALF), lambda i: (0, i, 0)),
            pl.BlockSpec((8, D), lambda i: (0, 0)),
            pl.BlockSpec((1, D), lambda i: (0, 0)),
            pl.BlockSpec((1, D), lambda i: (0, 0)),
        ],
        out_specs=pl.BlockSpec((_BN, D), lambda i: (i, 0)),
        out_shape=jax.ShapeDtypeStruct((N, D), jnp.float32),
    )(agg, st, g, b)


def kernel(x, edge_index, edge_attr, We, Win, b_in, Wc, bc, gamma, beta):
    del bc
    E = edge_index.shape[1]
    L = Wc.shape[0]
    src = edge_index[0]
    dst = edge_index[1]

    eaT = jnp.pad(edge_attr.T, ((0, 0), (0, EPAD - E))).reshape(-1)
    dpre = jnp.concatenate([dst, jnp.full((EPAD - E,), N, jnp.int32)])
    wep = jnp.pad(We[:, 0], (0, 13))
    ewp, degp = _sc_pre(eaT, dpre, wep)
    ew = ewp[:E]

    loop = jnp.arange(N, dtype=jnp.int32)
    padi = jnp.zeros((ENPAD - E - N,), jnp.int32)
    s2p = jnp.concatenate([src, loop, padi])
    d2p = jnp.concatenate([dst, loop, padi])
    w2p = jnp.concatenate(
        [ew, jnp.ones((N,), jnp.float32), jnp.zeros((ENPAD - E - N,), jnp.float32)])
    normp = _sc_norm(degp, s2p, d2p, w2p)
    s2pp = jnp.concatenate([s2p, s2p + N])

    b2 = b_in.reshape(1, D)
    mh = _tc_input(x, Win, b2, Wc[0])
    for i in range(L):
        agg2 = _sc_layer(mh.reshape(2 * N, HALF), s2pp, d2p, normp)
        agg = agg2.reshape(2, N, HALF)
        st = _tc_stats(agg)
        g = gamma[i].reshape(1, D)
        bt = beta[i].reshape(1, D)
        if i < L - 1:
            mh = _tc_mid(agg, st, g, bt, Wc[i + 1])
        else:
            out = _tc_final(agg, st, g, bt)
    return out


# speedup vs baseline: 3.0356x; 1.0223x over previous
import functools

import jax
import jax.numpy as jnp
from jax import lax
from jax.experimental import pallas as pl
from jax.experimental.pallas import tpu as pltpu
from jax.experimental.pallas import tpu_sc as plsc

N = 10000
D = 256
HALF = 128
NC = 2
NS = 16
LANES = 16

E_RAW = 160000
EPAD = 163840
EN = E_RAW + N
ENPAD = 172032

_mesh = plsc.VectorSubcoreMesh(core_axis_name="c", subcore_axis_name="s")

MAGIC = 0x5F3759DF


def _rsqrt_newton(x):
    i = lax.bitcast_convert_type(x, jnp.int32)
    i = MAGIC - lax.shift_right_arithmetic(i, 1)
    y = lax.bitcast_convert_type(i, jnp.float32)
    for _ in range(3):
        y = y * (1.5 - 0.5 * x * y * y)
    return y


_EPP = EPAD // (NC * NS)
_NPT = ENPAD // (NC * NS)

@functools.partial(
    pl.kernel,
    out_type=(
        jax.ShapeDtypeStruct((EPAD,), jnp.float32),
        jax.ShapeDtypeStruct((2 * N,), jnp.float32),
    ),
    mesh=_mesh,
    scratch_types=(
        pltpu.VMEM((_EPP,), jnp.float32),
        pltpu.VMEM((_EPP,), jnp.float32),
        pltpu.VMEM((_EPP,), jnp.float32),
        pltpu.VMEM((_EPP,), jnp.float32),
        pltpu.VMEM((_EPP,), jnp.int32),
        pltpu.VMEM((_EPP // 128, 128), jnp.int32),
        pltpu.VMEM((16,), jnp.float32),
        pltpu.VMEM((640,), jnp.float32),
        pltpu.VMEM_SHARED((N + 16,), jnp.float32),
    ),
)
def _sc_pre(ea_hbm, d_hbm, we_hbm, ew_hbm, deg_hbm,
            a0, a1, a2, ewv, dv1, dv2, wev, zbuf, degacc):
    cid = lax.axis_index("c")
    sid = lax.axis_index("s")

    def zstep(i, _):
        zbuf[pl.ds(i * 16, 16)] = jnp.zeros((16,), jnp.float32)
        return 0
    lax.fori_loop(0, 40, zstep, 0)

    @pl.when(sid < 15)
    def _():
        pltpu.sync_copy(zbuf, degacc.at[pl.ds(sid * 640, 640)])

    @pl.when(sid == 15)
    def _():
        pltpu.sync_copy(zbuf.at[pl.ds(0, 416)], degacc.at[pl.ds(9600, 416)])

    plsc.subcore_barrier()

    pltpu.sync_copy(we_hbm, wev)
    wvec = wev[...]
    w0 = wvec[0]
    w1 = wvec[1]
    w2 = wvec[2]
    base = (cid * NS + sid) * _EPP
    pltpu.sync_copy(ea_hbm.at[pl.ds(base, _EPP)], a0)
    pltpu.sync_copy(ea_hbm.at[pl.ds(EPAD + base, _EPP)], a1)
    pltpu.sync_copy(ea_hbm.at[pl.ds(2 * EPAD + base, _EPP)], a2)
    pltpu.sync_copy(d_hbm.at[pl.ds(base, _EPP)], dv1)

    def rpk(j, _):
        for c in range(8):
            dv2[j, pl.ds(c * 16, 16)] = dv1[pl.ds(j * 128 + c * 16, 16)]
        return 0
    lax.fori_loop(0, _EPP // 128, rpk, 0)

    def cstep(j, _):
        sl = pl.ds(j * 16, 16)
        z = a0[sl] * w0 + a1[sl] * w1 + a2[sl] * w2
        ewv[sl] = 1.0 / (1.0 + jnp.exp(-z))
        return 0
    lax.fori_loop(0, _EPP // 16, cstep, 0)

    def sct(b, _):
        pltpu.sync_copy(ewv.at[pl.ds(b * 128, 128)],
                        degacc.at[dv2.at[b]], add=True)
        return 0
    lax.fori_loop(0, _EPP // 128, sct, 0)
    pltpu.sync_copy(ewv, ew_hbm.at[pl.ds(base, _EPP)])

    plsc.subcore_barrier()

    @pl.when(sid < 15)
    def _():
        pltpu.sync_copy(degacc.at[pl.ds(sid * 640, 640)], zbuf)
        pltpu.sync_copy(zbuf, deg_hbm.at[pl.ds(cid * N + sid * 640, 640)])

    @pl.when(sid == 15)
    def _():
        pltpu.sync_copy(degacc.at[pl.ds(9600, 400)], zbuf.at[pl.ds(0, 400)])
        pltpu.sync_copy(zbuf.at[pl.ds(0, 400)],
                        deg_hbm.at[pl.ds(cid * N + 9600, 400)])


@functools.partial(
    pl.kernel,
    out_type=jax.ShapeDtypeStruct((ENPAD,), jnp.float32),
    mesh=_mesh,
    compiler_params=pltpu.CompilerParams(needs_layout_passes=False),
    scratch_types=(
        pltpu.VMEM((N,), jnp.float32),
        pltpu.VMEM((N,), jnp.float32),
        pltpu.VMEM((_NPT,), jnp.int32),
        pltpu.VMEM((_NPT,), jnp.int32),
        pltpu.VMEM((_NPT,), jnp.float32),
        pltpu.VMEM((_NPT,), jnp.float32),
    ),
)
def _sc_norm(deg_hbm, s_hbm, d_hbm, w_hbm, norm_hbm,
             degv, tmpv, sv, dv, wv, nv):
    cid = lax.axis_index("c")
    sid = lax.axis_index("s")
    pltpu.sync_copy(deg_hbm.at[pl.ds(0, N)], degv)
    pltpu.sync_copy(deg_hbm.at[pl.ds(N, N)], tmpv)

    def astep(i, _):
        sl = pl.ds(i * 16, 16)
        degv[sl] = degv[sl] + tmpv[sl] + 1.0
        return 0
    lax.fori_loop(0, N // 16, astep, 0)

    base = (cid * NS + sid) * _NPT
    pltpu.sync_copy(s_hbm.at[pl.ds(base, _NPT)], sv)
    pltpu.sync_copy(d_hbm.at[pl.ds(base, _NPT)], dv)
    pltpu.sync_copy(w_hbm.at[pl.ds(base, _NPT)], wv)

    def cstep(j, _):
        sl = pl.ds(j * 16, 16)
        dg = plsc.load_gather(degv, [sv[sl]])
        dd = plsc.load_gather(degv, [dv[sl]])
        nv[sl] = wv[sl] * _rsqrt_newton(dg * dd)
        return 0
    lax.fori_loop(0, _NPT // 16, cstep, 0)
    pltpu.sync_copy(nv, norm_hbm.at[pl.ds(base, _NPT)])


_EPT = ENPAD // NS
_SUP = _EPT // 3
_SC_ROWS = 128
_NSUB = _SUP // _SC_ROWS
_RING = 2
_NQUAD = _NSUB // _RING

@functools.partial(
    pl.kernel,
    out_type=jax.ShapeDtypeStruct((2 * N, HALF), jnp.float32),
    mesh=_mesh,
    scratch_types=(
        tuple(pltpu.VMEM((_SC_ROWS, HALF), jnp.float32) for _ in range(_RING)),
        pltpu.VMEM((_SUP,), jnp.int32),
        pltpu.VMEM((_SUP,), jnp.int32),
        pltpu.VMEM((_NSUB, _SC_ROWS), jnp.int32),
        pltpu.VMEM((_SUP,), jnp.float32),
        pltpu.VMEM_SHARED((N, HALF), jnp.float32),
        tuple(pltpu.SemaphoreType.DMA for _ in range(_RING)),
        tuple(pltpu.SemaphoreType.DMA for _ in range(_RING)),
    ),
)
def _sc_layer(m_hbm, s_hbm, d_hbm, norm_hbm, agg_hbm,
              sbufs, sbig, dbig, dv2, nbig, acc, gsems, ssems):
    cid = lax.axis_index("c")
    sid = lax.axis_index("s")

    def zrow(e, _):
        for j in range(8):
            sbufs[0][e, pl.ds(j * 16, 16)] = jnp.zeros((16,), jnp.float32)
        return 0
    lax.fori_loop(0, _SC_ROWS, zrow, 0)
    roff = pl.multiple_of(sid * 624, 8)
    for h in range(4):
        pltpu.sync_copy(sbufs[0], acc.at[pl.ds(roff + h * 128, 128)])

    @pl.when(sid < 15)
    def _():
        pltpu.sync_copy(sbufs[0].at[pl.ds(0, 112)],
                        acc.at[pl.ds(roff + 512, 112)])

    @pl.when(sid == 15)
    def _():
        pltpu.sync_copy(sbufs[0], acc.at[pl.ds(roff + 512, 128)])
    plsc.subcore_barrier()

    base = sid * _EPT

    def scale(l, noff):
        sbuf = sbufs[l]

        def sgroup(g, _):
            nvec = nbig[pl.ds(noff + g * 16, 16)]
            for c in range(16):
                s = nvec[c]
                e = g * 16 + c
                for j in range(8):
                    sl = pl.ds(j * 16, 16)
                    sbuf[e, sl] = sbuf[e, sl] * s
            return 0
        lax.fori_loop(0, _SC_ROWS // 16, sgroup, 0)

    def gather(b, l):
        pltpu.async_copy(
            m_hbm.at[sbig.at[pl.ds(b * _SC_ROWS, _SC_ROWS)]], sbufs[l],
            gsems[l])

    def gwait(l):
        pltpu.make_async_copy(
            m_hbm.at[sbig.at[pl.ds(0, _SC_ROWS)]], sbufs[l], gsems[l]).wait()

    def swait(l):
        pltpu.make_async_copy(sbufs[l], acc.at[dv2.at[0]], ssems[l]).wait()

    def superchunk(s, _):
        sbase = base + s * _SUP
        pltpu.sync_copy(s_hbm.at[pl.ds(cid * ENPAD + sbase, _SUP)], sbig)
        pltpu.sync_copy(d_hbm.at[pl.ds(sbase, _SUP)], dbig)
        pltpu.sync_copy(norm_hbm.at[pl.ds(sbase, _SUP)], nbig)

        def repack(j, _):
            for c in range(_SC_ROWS // 16):
                dv2[j, pl.ds(c * 16, 16)] = dbig[
                    pl.ds(j * _SC_ROWS + c * 16, 16)]
            return 0
        lax.fori_loop(0, _NSUB, repack, 0)

        for l in range(_RING):
            gather(l, l)

        def quad(q, _):
            for l in range(_RING):
                b = _RING * q + l
                gwait(l)
                scale(l, b * _SC_ROWS)
                pltpu.async_copy(sbufs[l], acc.at[dv2.at[b]], ssems[l],
                                 add=True)

            @pl.when(q < _NQUAD - 1)
            def _():
                for l in range(_RING):
                    swait(l)
                    gather(_RING * (q + 1) + l, l)
            return 0
        lax.fori_loop(0, _NQUAD, quad, 0)

        for l in range(_RING):
            swait(l)
        return 0
    lax.fori_loop(0, _EPT // _SUP, superchunk, 0)

    plsc.subcore_barrier()
    hoff = pl.multiple_of(cid * N + sid * 624, 8)
    for h in range(4):
        b = sbufs[h % 2]
        pltpu.sync_copy(acc.at[pl.ds(roff + h * 128, 128)], b)
        pltpu.sync_copy(b, agg_hbm.at[pl.ds(hoff + h * 128, 128)])

    @pl.when(sid < 15)
    def _():
        pltpu.sync_copy(acc.at[pl.ds(roff + 512, 112)],
                        sbufs[1].at[pl.ds(0, 112)])
        pltpu.sync_copy(sbufs[1].at[pl.ds(0, 112)],
                        agg_hbm.at[pl.ds(hoff + 512, 112)])

    @pl.when(sid == 15)
    def _():
        pltpu.sync_copy(acc.at[pl.ds(roff + 512, 128)], sbufs[1])
        pltpu.sync_copy(sbufs[1], agg_hbm.at[pl.ds(hoff + 512, 128)])


_BN = 1000
_GRID = N // _BN


def _dot(a, b):
    return jnp.dot(a, b, precision=jax.lax.Precision.HIGHEST,
                   preferred_element_type=jnp.float32)


def _tc_input(x, win, b_in, wc0):
    def body(x_ref, w_ref, b_ref, wc_ref, o_ref):
        h = jnp.maximum(_dot(x_ref[:], w_ref[:]) + b_ref[:], 0.0)
        m = _dot(h, wc_ref[:])
        o_ref[0] = m[:, :HALF]
        o_ref[1] = m[:, HALF:]

    return pl.pallas_call(
        body,
        grid=(_GRID,),
        in_specs=[
            pl.BlockSpec((_BN, D), lambda i: (i, 0)),
            pl.BlockSpec((D, D), lambda i: (0, 0)),
            pl.BlockSpec((1, D), lambda i: (0, 0)),
            pl.BlockSpec((D, D), lambda i: (0, 0)),
        ],
        out_specs=pl.BlockSpec((2, _BN, HALF), lambda i: (0, i, 0)),
        out_shape=jax.ShapeDtypeStruct((2, N, HALF), jnp.float32),
    )(x, win, b_in, wc0)


def _tc_stats(agg):
    def body(a_ref, o_ref):
        i = pl.program_id(0)
        a = jnp.concatenate([a_ref[0], a_ref[1]], axis=1)
        s1 = jnp.sum(a, axis=0, keepdims=True)
        s2 = jnp.sum(a * a, axis=0, keepdims=True)
        blk = jnp.concatenate(
            [s1, s2, jnp.zeros((6, D), jnp.float32)], axis=0)

        @pl.when(i == 0)
        def _():
            o_ref[:] = blk

        @pl.when(i > 0)
        def _():
            o_ref[:] = o_ref[:] + blk

    return pl.pallas_call(
        body,
        grid=(_GRID,),
        in_specs=[pl.BlockSpec((2, _BN, HALF), lambda i: (0, i, 0))],
        out_specs=pl.BlockSpec((8, D), lambda i: (0, 0)),
        out_shape=jax.ShapeDtypeStruct((8, D), jnp.float32),
    )(agg)


def _bn_relu(a_ref, st_ref, g_ref, b_ref):
    a = jnp.concatenate([a_ref[0], a_ref[1]], axis=1)
    mean = st_ref[0:1, :] * (1.0 / N)
    ex2 = st_ref[1:2, :] * (1.0 / N)
    var = ex2 - mean * mean
    inv = lax.rsqrt(var + 1e-5)
    return jnp.maximum((a - mean) * inv * g_ref[:] + b_ref[:], 0.0)


def _tc_mid(agg, st, g, b, wc):
    def body(a_ref, st_ref, g_ref, b_ref, wc_ref, o_ref):
        h = _bn_relu(a_ref, st_ref, g_ref, b_ref)
        m = _dot(h, wc_ref[:])
        o_ref[0] = m[:, :HALF]
        o_ref[1] = m[:, HALF:]

    return pl.pallas_call(
        body,
        grid=(_GRID,),
        in_specs=[
            pl.BlockSpec((2, _BN, HALF), lambda i: (0, i, 0)),
            pl.BlockSpec((8, D), lambda i: (0, 0)),
            pl.BlockSpec((1, D), lambda i: (0, 0)),
            pl.BlockSpec((1, D), lambda i: (0, 0)),
            pl.BlockSpec((D, D), lambda i: (0, 0)),
        ],
        out_specs=pl.BlockSpec((2, _BN, HALF), lambda i: (0, i, 0)),
        out_shape=jax.ShapeDtypeStruct((2, N, HALF), jnp.float32),
    )(agg, st, g, b, wc)


def _tc_final(agg, st, g, b):
    def body(a_ref, st_ref, g_ref, b_ref, o_ref):
        o_ref[:] = _bn_relu(a_ref, st_ref, g_ref, b_ref)

    return pl.pallas_call(
        body,
        grid=(_GRID,),
        in_specs=[
            pl.BlockSpec((2, _BN, HALF), lambda i: (0, i, 0)),
            pl.BlockSpec((8, D), lambda i: (0, 0)),
            pl.BlockSpec((1, D), lambda i: (0, 0)),
            pl.BlockSpec((1, D), lambda i: (0, 0)),
        ],
        out_specs=pl.BlockSpec((_BN, D), lambda i: (i, 0)),
        out_shape=jax.ShapeDtypeStruct((N, D), jnp.float32),
    )(agg, st, g, b)


def kernel(x, edge_index, edge_attr, We, Win, b_in, Wc, bc, gamma, beta):
    del bc
    E = edge_index.shape[1]
    L = Wc.shape[0]
    src = edge_index[0]
    dst = edge_index[1]

    eaT = jnp.pad(edge_attr.T, ((0, 0), (0, EPAD - E))).reshape(-1)
    dpre = jnp.concatenate([dst, jnp.full((EPAD - E,), N, jnp.int32)])
    wep = jnp.pad(We[:, 0], (0, 13))
    ewp, degp = _sc_pre(eaT, dpre, wep)
    ew = ewp[:E]

    loop = jnp.arange(N, dtype=jnp.int32)
    padi = jnp.zeros((ENPAD - E - N,), jnp.int32)
    s2p = jnp.concatenate([src, loop, padi])
    d2p = jnp.concatenate([dst, loop, padi])
    w2p = jnp.concatenate(
        [ew, jnp.ones((N,), jnp.float32), jnp.zeros((ENPAD - E - N,), jnp.float32)])
    normp = _sc_norm(degp, s2p, d2p, w2p)
    s2pp = jnp.concatenate([s2p, s2p + N])

    b2 = b_in.reshape(1, D)
    mh = _tc_input(x, Win, b2, Wc[0])
    for i in range(L):
        agg2 = _sc_layer(mh.reshape(2 * N, HALF), s2pp, d2p, normp)
        agg = agg2.reshape(2, N, HALF)
        st = _tc_stats(agg)
        g = gamma[i].reshape(1, D)
        bt = beta[i].reshape(1, D)
        if i < L - 1:
            mh = _tc_mid(agg, st, g, bt, Wc[i + 1])
        else:
            out = _tc_final(agg, st, g, bt)
    return out
